# baseline (speedup 1.0000x reference)
import functools

import jax
import jax.numpy as jnp
from jax import lax
from jax.experimental import pallas as pl
from jax.experimental.pallas import tpu as pltpu
from jax.experimental.pallas import tpu_sc as plsc

N = 50000
CI = 128
CO = 128
K = 6

NC = 2
NS = 16
NWORK = NC * NS
NPW = 1568
NPAD = NWORK * NPW
CHUNK = 56
NCHUNK = NPW // CHUNK
BN = 1024
GRID = NPAD // BN
EPS = 1e-5


def _y_index(i, j):
    return (jnp.where(j == 0, i, (j - 1) * GRID + i), 0)


def _proj1_body(x_ref, w_ref, b_ref, y_ref, s_ref):
    j = pl.program_id(1)
    xb = x_ref[...].astype(jnp.bfloat16)
    r = lax.dot_general(xb, w_ref[...], (((0,), (0,)), ((), ())),
                        preferred_element_type=jnp.float32)
    y_ref[...] = r

    @pl.when(j == 0)
    def _():
        s_ref[...] = r + b_ref[...]


def _proj1(x_pad, w1cat, b1r):
    return pl.pallas_call(
        _proj1_body,
        grid=(GRID, K + 1),
        in_specs=[
            pl.BlockSpec((CI, BN), lambda i, j: (0, i)),
            pl.BlockSpec((CI, CO), lambda i, j: (0, j)),
            pl.BlockSpec((1, CO), lambda i, j: (0, 0)),
        ],
        out_specs=[
            pl.BlockSpec((BN, CO), _y_index),
            pl.BlockSpec((BN, CO), lambda i, j: (i, 0)),
        ],
        out_shape=[
            jax.ShapeDtypeStruct((K * NPAD, CO), jnp.float32),
            jax.ShapeDtypeStruct((NPAD, CO), jnp.float32),
        ],
    )(x_pad, w1cat, b1r)


def _proj2_body(z1_ref, xd_ref, wa_ref, wb_ref, b_ref, y_ref, s_ref):
    j = pl.program_id(1)
    x1 = z1_ref[...].astype(jnp.bfloat16)
    xd = xd_ref[...].astype(jnp.bfloat16)
    r = lax.dot_general(x1, wa_ref[...], (((1,), (0,)), ((), ())),
                        preferred_element_type=jnp.float32)
    r = r + lax.dot_general(xd, wb_ref[...], (((0,), (0,)), ((), ())),
                            preferred_element_type=jnp.float32)
    y_ref[...] = r

    @pl.when(j == 0)
    def _():
        s_ref[...] = r + b_ref[...]


def _proj2(z1, xd_pad, w2a, w2b, b2r):
    return pl.pallas_call(
        _proj2_body,
        grid=(GRID, K + 1),
        in_specs=[
            pl.BlockSpec((BN, CO), lambda i, j: (i, 0)),
            pl.BlockSpec((CO, BN), lambda i, j: (0, i)),
            pl.BlockSpec((CO, CO), lambda i, j: (0, j)),
            pl.BlockSpec((CO, CO), lambda i, j: (0, j)),
            pl.BlockSpec((1, CO), lambda i, j: (0, 0)),
        ],
        out_specs=[
            pl.BlockSpec((BN, CO), _y_index),
            pl.BlockSpec((BN, CO), lambda i, j: (i, 0)),
        ],
        out_shape=[
            jax.ShapeDtypeStruct((K * NPAD, CO), jnp.float32),
            jax.ShapeDtypeStruct((NPAD, CO), jnp.float32),
        ],
    )(z1, xd_pad, w2a, w2b, b2r)


NK16 = CO // 16


def _make_gather(with_stats):
    def body(table_hbm, idx_hbm, s_hbm, *rest):
        if with_stats:
            (out_hbm, parts_hbm, idx_v, buf, sbuf, obuf, stats_v,
             semg0, semg1, sems0, sems1) = rest
        else:
            (out_hbm, idx_v, buf, sbuf, obuf,
             semg0, semg1, sems0, sems1) = rest
        wid = lax.axis_index("s") * NC + lax.axis_index("c")
        base_chunk = wid * NCHUNK
        semg = (semg0, semg1)
        sems = (sems0, sems1)

        def fire(b, g):
            pltpu.sync_copy(idx_hbm.at[base_chunk + g], idx_v.at[b])
            for t in range(K):
                pltpu.async_copy(table_hbm.at[idx_v.at[b, t]], buf.at[b, t],
                                 semg[b])
            pltpu.async_copy(
                s_hbm.at[pl.ds((base_chunk + g) * CHUNK, CHUNK)],
                sbuf.at[b], semg[b])

        def wait_gathers(b):
            for t in range(K):
                pltpu.make_async_copy(table_hbm.at[pl.ds(0, CHUNK)],
                                      buf.at[b, t], semg[b]).wait()
            pltpu.make_async_copy(s_hbm.at[pl.ds(0, CHUNK)],
                                  sbuf.at[b], semg[b]).wait()

        def accumulate(b, g, st):
            def row_body(r, stc):
                out = list(stc) if with_stats else stc
                for k in range(NK16):
                    sl = pl.ds(k * 16, 16)
                    acc = sbuf[b, r, sl] + buf[b, 0, r, sl]
                    for t in range(1, K):
                        acc = acc + buf[b, t, r, sl]
                    obuf[b, r, sl] = acc
                    if with_stats:
                        valid = ((base_chunk + g) * CHUNK + r) < N
                        av = jnp.where(valid, acc, 0.0)
                        out[k] = out[k] + av
                        out[NK16 + k] = out[NK16 + k] + av * av
                return tuple(out) if with_stats else stc

            return lax.fori_loop(0, CHUNK, row_body, st)

        def consume(b, i, st):
            g = 2 * i + b
            wait_gathers(b)

            @pl.when(i > 0)
            def _():
                pltpu.make_async_copy(obuf.at[b],
                                      out_hbm.at[pl.ds(0, CHUNK)],
                                      sems[b]).wait()

            st = accumulate(b, g, st)
            pltpu.async_copy(
                obuf.at[b],
                out_hbm.at[pl.ds((base_chunk + g) * CHUNK, CHUNK)],
                sems[b])

            @pl.when(i < NCHUNK // 2 - 1)
            def _():
                fire(b, g + 2)

            return st

        fire(0, 0)
        fire(1, 1)
        init = (tuple(jnp.zeros((16,), jnp.float32) for _ in range(2 * NK16))
                if with_stats else 0)

        def pair_body(i, st):
            st = consume(0, i, st)
            st = consume(1, i, st)
            return st

        st = lax.fori_loop(0, NCHUNK // 2, pair_body, init)
        for b in range(2):
            pltpu.make_async_copy(obuf.at[b], out_hbm.at[pl.ds(0, CHUNK)],
                                  sems[b]).wait()
        if with_stats:
            for k in range(NK16):
                stats_v[0, pl.ds(k * 16, 16)] = st[k]
                stats_v[1, pl.ds(k * 16, 16)] = st[NK16 + k]
            pltpu.sync_copy(stats_v, parts_hbm.at[wid])

    out_type = jax.ShapeDtypeStruct((NPAD, CO), jnp.float32)
    scratch = [
        pltpu.VMEM((2, K, CHUNK), jnp.int32),
        pltpu.VMEM((2, K, CHUNK, CO), jnp.float32),
        pltpu.VMEM((2, CHUNK, CO), jnp.float32),
        pltpu.VMEM((2, CHUNK, CO), jnp.float32),
    ]
    if with_stats:
        out_type = [out_type,
                    jax.ShapeDtypeStruct((NWORK, 2, CO), jnp.float32)]
        scratch = scratch + [pltpu.VMEM((2, CO), jnp.float32)]
    scratch = scratch + [pltpu.SemaphoreType.DMA] * 4
    return functools.partial(
        pl.kernel,
        mesh=plsc.VectorSubcoreMesh(core_axis_name="c", subcore_axis_name="s"),
        out_type=out_type,
        scratch_types=scratch,
    )(body)


_gather_plain = _make_gather(False)
_gather_stats = _make_gather(True)


def _norm_body(z_ref, p_ref, o_ref):
    z = z_ref[...]
    ssum = jnp.sum(p_ref[:, 0, :], axis=0)
    sqsum = jnp.sum(p_ref[:, 1, :], axis=0)
    mean = ssum * (1.0 / N)
    var = sqsum * (1.0 / N) - mean * mean
    inv = lax.rsqrt(var + EPS)
    zn = (z - mean[None, :]) * inv[None, :]
    o_ref[...] = zn.T[None]


def _norm(z2, parts):
    return pl.pallas_call(
        _norm_body,
        grid=(GRID,),
        in_specs=[
            pl.BlockSpec((BN, CO), lambda i: (i, 0)),
            pl.BlockSpec((NWORK, 2, CO), lambda i: (0, 0, 0)),
        ],
        out_specs=pl.BlockSpec((1, CO, BN), lambda i: (0, 0, i)),
        out_shape=jax.ShapeDtypeStruct((1, CO, N), jnp.float32),
    )(z2, parts)


def kernel(from_up, from_down, neighbors, W1, b1, W2, b2):
    f32 = jnp.float32
    xu = from_up[0]
    xd = from_down[0]

    bf16 = jnp.bfloat16
    w1cat = W1.transpose(1, 2, 0).reshape(CI, (K + 1) * CO).astype(bf16)
    w2a = W2[:, :CO, :].transpose(1, 2, 0).reshape(CO, (K + 1) * CO).astype(bf16)
    w2b = W2[:, CO:, :].transpose(1, 2, 0).reshape(CO, (K + 1) * CO).astype(bf16)
    b1r = b1[None, :].astype(f32)
    b2r = b2[None, :].astype(f32)

    nbr_pad = jnp.concatenate(
        [neighbors.astype(jnp.int32),
         jnp.zeros((NPAD - N, K), jnp.int32)], axis=0)
    adj = nbr_pad + (jnp.arange(K, dtype=jnp.int32) * NPAD)[None, :]
    idxarr = adj.reshape(NWORK * NCHUNK, CHUNK, K).transpose(0, 2, 1)

    y1, s1 = _proj1(xu, w1cat, b1r)
    z1 = _gather_plain(y1, idxarr, s1)
    y2, s2 = _proj2(z1, xd, w2a, w2b, b2r)
    z2, parts = _gather_stats(y2, idxarr, s2)
    return _norm(z2, parts)


# speedup vs baseline: 1.7019x; 1.7019x over previous
import functools

import jax
import jax.numpy as jnp
from jax import lax
from jax.experimental import pallas as pl
from jax.experimental.pallas import tpu as pltpu
from jax.experimental.pallas import tpu_sc as plsc

N = 50000
CI = 128
CO = 128
K = 6

NC = 2
NS = 16
NWORK = NC * NS
NPW = 1568
NPAD = NWORK * NPW
CHUNK = 56
NCHUNK = NPW // CHUNK
BN = 1024
GRID = NPAD // BN
EPS = 1e-5


def _proj1_body(x_ref, w_ref, b_ref, y_ref, s_ref):
    xb = x_ref[...].astype(jnp.bfloat16)
    for j in range(K + 1):
        w = w_ref[:, j * CO:(j + 1) * CO]
        r = lax.dot_general(xb, w, (((0,), (0,)), ((), ())),
                            preferred_element_type=jnp.float32)
        if j == 0:
            s_ref[...] = r + b_ref[...]
        else:
            y_ref[j - 1, :, :] = r


def _proj1(x_pad, w1cat, b1r):
    return pl.pallas_call(
        _proj1_body,
        grid=(GRID,),
        in_specs=[
            pl.BlockSpec((CI, BN), lambda i: (0, i)),
            pl.BlockSpec((CI, (K + 1) * CO), lambda i: (0, 0)),
            pl.BlockSpec((1, CO), lambda i: (0, 0)),
        ],
        out_specs=[
            pl.BlockSpec((K, BN, CO), lambda i: (0, i, 0)),
            pl.BlockSpec((BN, CO), lambda i: (i, 0)),
        ],
        out_shape=[
            jax.ShapeDtypeStruct((K, NPAD, CO), jnp.float32),
            jax.ShapeDtypeStruct((NPAD, CO), jnp.float32),
        ],
    )(x_pad, w1cat, b1r)


def _proj2_body(z1_ref, xd_ref, wa_ref, wb_ref, b_ref, y_ref, s_ref):
    x1 = z1_ref[...].astype(jnp.bfloat16)
    xd = xd_ref[...].astype(jnp.bfloat16)
    for j in range(K + 1):
        wa = wa_ref[:, j * CO:(j + 1) * CO]
        wb = wb_ref[:, j * CO:(j + 1) * CO]
        r = lax.dot_general(x1, wa, (((1,), (0,)), ((), ())),
                            preferred_element_type=jnp.float32)
        r = r + lax.dot_general(xd, wb, (((0,), (0,)), ((), ())),
                                preferred_element_type=jnp.float32)
        if j == 0:
            s_ref[...] = r + b_ref[...]
        else:
            y_ref[j - 1, :, :] = r


def _proj2(z1, xd_pad, w2a, w2b, b2r):
    return pl.pallas_call(
        _proj2_body,
        grid=(GRID,),
        in_specs=[
            pl.BlockSpec((BN, CO), lambda i: (i, 0)),
            pl.BlockSpec((CO, BN), lambda i: (0, i)),
            pl.BlockSpec((CO, (K + 1) * CO), lambda i: (0, 0)),
            pl.BlockSpec((CO, (K + 1) * CO), lambda i: (0, 0)),
            pl.BlockSpec((1, CO), lambda i: (0, 0)),
        ],
        out_specs=[
            pl.BlockSpec((K, BN, CO), lambda i: (0, i, 0)),
            pl.BlockSpec((BN, CO), lambda i: (i, 0)),
        ],
        out_shape=[
            jax.ShapeDtypeStruct((K, NPAD, CO), jnp.float32),
            jax.ShapeDtypeStruct((NPAD, CO), jnp.float32),
        ],
    )(z1, xd_pad, w2a, w2b, b2r)


NK16 = CO // 16


def _make_gather(with_stats):
    def body(table_hbm, idx_hbm, s_hbm, *rest):
        if with_stats:
            (out_hbm, parts_hbm, idx_v, buf, sbuf, obuf, stats_v,
             semg0, semg1, sems0, sems1) = rest
        else:
            (out_hbm, idx_v, buf, sbuf, obuf,
             semg0, semg1, sems0, sems1) = rest
        wid = lax.axis_index("s") * NC + lax.axis_index("c")
        base_chunk = wid * NCHUNK
        semg = (semg0, semg1)
        sems = (sems0, sems1)

        def fire(b, g):
            pltpu.sync_copy(idx_hbm.at[base_chunk + g], idx_v.at[b])
            for t in range(K):
                pltpu.async_copy(table_hbm.at[idx_v.at[b, t]], buf.at[b, t],
                                 semg[b])
            pltpu.async_copy(
                s_hbm.at[pl.ds((base_chunk + g) * CHUNK, CHUNK)],
                sbuf.at[b], semg[b])

        def wait_gathers(b):
            for t in range(K):
                pltpu.make_async_copy(table_hbm.at[pl.ds(0, CHUNK)],
                                      buf.at[b, t], semg[b]).wait()
            pltpu.make_async_copy(s_hbm.at[pl.ds(0, CHUNK)],
                                  sbuf.at[b], semg[b]).wait()

        def accumulate(b, g, st):
            def row_body(r, stc):
                out = list(stc) if with_stats else stc
                for k in range(NK16):
                    sl = pl.ds(k * 16, 16)
                    acc = sbuf[b, r, sl] + buf[b, 0, r, sl]
                    for t in range(1, K):
                        acc = acc + buf[b, t, r, sl]
                    obuf[b, r, sl] = acc
                    if with_stats:
                        valid = ((base_chunk + g) * CHUNK + r) < N
                        av = jnp.where(valid, acc, 0.0)
                        out[k] = out[k] + av
                        out[NK16 + k] = out[NK16 + k] + av * av
                return tuple(out) if with_stats else stc

            return lax.fori_loop(0, CHUNK, row_body, st)

        def consume(b, i, st):
            g = 2 * i + b
            wait_gathers(b)

            @pl.when(i > 0)
            def _():
                pltpu.make_async_copy(obuf.at[b],
                                      out_hbm.at[pl.ds(0, CHUNK)],
                                      sems[b]).wait()

            st = accumulate(b, g, st)
            pltpu.async_copy(
                obuf.at[b],
                out_hbm.at[pl.ds((base_chunk + g) * CHUNK, CHUNK)],
                sems[b])

            @pl.when(i < NCHUNK // 2 - 1)
            def _():
                fire(b, g + 2)

            return st

        fire(0, 0)
        fire(1, 1)
        init = (tuple(jnp.zeros((16,), jnp.float32) for _ in range(2 * NK16))
                if with_stats else 0)

        def pair_body(i, st):
            st = consume(0, i, st)
            st = consume(1, i, st)
            return st

        st = lax.fori_loop(0, NCHUNK // 2, pair_body, init)
        for b in range(2):
            pltpu.make_async_copy(obuf.at[b], out_hbm.at[pl.ds(0, CHUNK)],
                                  sems[b]).wait()
        if with_stats:
            for k in range(NK16):
                stats_v[0, pl.ds(k * 16, 16)] = st[k]
                stats_v[1, pl.ds(k * 16, 16)] = st[NK16 + k]
            pltpu.sync_copy(stats_v, parts_hbm.at[wid])

    out_type = jax.ShapeDtypeStruct((NPAD, CO), jnp.float32)
    scratch = [
        pltpu.VMEM((2, K, CHUNK), jnp.int32),
        pltpu.VMEM((2, K, CHUNK, CO), jnp.float32),
        pltpu.VMEM((2, CHUNK, CO), jnp.float32),
        pltpu.VMEM((2, CHUNK, CO), jnp.float32),
    ]
    if with_stats:
        out_type = [out_type,
                    jax.ShapeDtypeStruct((NWORK, 2, CO), jnp.float32)]
        scratch = scratch + [pltpu.VMEM((2, CO), jnp.float32)]
    scratch = scratch + [pltpu.SemaphoreType.DMA] * 4
    return functools.partial(
        pl.kernel,
        mesh=plsc.VectorSubcoreMesh(core_axis_name="c", subcore_axis_name="s"),
        out_type=out_type,
        scratch_types=scratch,
    )(body)


_gather_plain = _make_gather(False)
_gather_stats = _make_gather(True)


def _norm_body(z_ref, p_ref, o_ref):
    z = z_ref[...]
    ssum = jnp.sum(p_ref[:, 0, :], axis=0)
    sqsum = jnp.sum(p_ref[:, 1, :], axis=0)
    mean = ssum * (1.0 / N)
    var = sqsum * (1.0 / N) - mean * mean
    inv = lax.rsqrt(var + EPS)
    zn = (z - mean[None, :]) * inv[None, :]
    o_ref[...] = zn.T[None]


def _norm(z2, parts):
    return pl.pallas_call(
        _norm_body,
        grid=(GRID,),
        in_specs=[
            pl.BlockSpec((BN, CO), lambda i: (i, 0)),
            pl.BlockSpec((NWORK, 2, CO), lambda i: (0, 0, 0)),
        ],
        out_specs=pl.BlockSpec((1, CO, BN), lambda i: (0, 0, i)),
        out_shape=jax.ShapeDtypeStruct((1, CO, N), jnp.float32),
    )(z2, parts)


def kernel(from_up, from_down, neighbors, W1, b1, W2, b2):
    f32 = jnp.float32
    xu = from_up[0]
    xd = from_down[0]

    bf16 = jnp.bfloat16
    w1cat = W1.transpose(1, 2, 0).reshape(CI, (K + 1) * CO).astype(bf16)
    w2a = W2[:, :CO, :].transpose(1, 2, 0).reshape(CO, (K + 1) * CO).astype(bf16)
    w2b = W2[:, CO:, :].transpose(1, 2, 0).reshape(CO, (K + 1) * CO).astype(bf16)
    b1r = b1[None, :].astype(f32)
    b2r = b2[None, :].astype(f32)

    nbr_pad = jnp.concatenate(
        [neighbors.astype(jnp.int32),
         jnp.zeros((NPAD - N, K), jnp.int32)], axis=0)
    adj = nbr_pad + (jnp.arange(K, dtype=jnp.int32) * NPAD)[None, :]
    idxarr = adj.reshape(NWORK * NCHUNK, CHUNK, K).transpose(0, 2, 1)

    y1, s1 = _proj1(xu, w1cat, b1r)
    z1 = _gather_plain(y1.reshape(K * NPAD, CO), idxarr, s1)
    y2, s2 = _proj2(z1, xd, w2a, w2b, b2r)
    z2, parts = _gather_stats(y2.reshape(K * NPAD, CO), idxarr, s2)
    return _norm(z2, parts)


# speedup vs baseline: 1.7057x; 1.0022x over previous
import functools

import jax
import jax.numpy as jnp
from jax import lax
from jax.experimental import pallas as pl
from jax.experimental.pallas import tpu as pltpu
from jax.experimental.pallas import tpu_sc as plsc

N = 50000
CI = 128
CO = 128
K = 6

NC = 2
NS = 16
NWORK = NC * NS
NPW = 1568
NPAD = NWORK * NPW
CHUNK = 56
NCHUNK = NPW // CHUNK
BN = 1024
GRID = NPAD // BN
EPS = 1e-5


def _proj1_body(x_ref, w_ref, b_ref, y_ref):
    xb = x_ref[...].astype(jnp.bfloat16)
    for j in range(K + 1):
        w = w_ref[:, j * CO:(j + 1) * CO]
        r = lax.dot_general(xb, w, (((0,), (0,)), ((), ())),
                            preferred_element_type=jnp.float32)
        if j == 0:
            y_ref[0, :, :] = r + b_ref[...]
        else:
            y_ref[j, :, :] = r


def _proj1(x_pad, w1cat, b1r):
    return pl.pallas_call(
        _proj1_body,
        grid=(GRID,),
        in_specs=[
            pl.BlockSpec((CI, BN), lambda i: (0, i)),
            pl.BlockSpec((CI, (K + 1) * CO), lambda i: (0, 0)),
            pl.BlockSpec((1, CO), lambda i: (0, 0)),
        ],
        out_specs=pl.BlockSpec((K + 1, BN, CO), lambda i: (0, i, 0)),
        out_shape=jax.ShapeDtypeStruct((K + 1, NPAD, CO), jnp.float32),
    )(x_pad, w1cat, b1r)


def _proj2_body(z1_ref, xd_ref, wa_ref, wb_ref, b_ref, y_ref):
    x1 = z1_ref[...].astype(jnp.bfloat16)
    xd = xd_ref[...].astype(jnp.bfloat16)
    for j in range(K + 1):
        wa = wa_ref[:, j * CO:(j + 1) * CO]
        wb = wb_ref[:, j * CO:(j + 1) * CO]
        r = lax.dot_general(x1, wa, (((1,), (0,)), ((), ())),
                            preferred_element_type=jnp.float32)
        r = r + lax.dot_general(xd, wb, (((0,), (0,)), ((), ())),
                                preferred_element_type=jnp.float32)
        if j == 0:
            y_ref[0, :, :] = r + b_ref[...]
        else:
            y_ref[j, :, :] = r


def _proj2(z1, xd_pad, w2a, w2b, b2r):
    return pl.pallas_call(
        _proj2_body,
        grid=(GRID,),
        in_specs=[
            pl.BlockSpec((BN, CO), lambda i: (i, 0)),
            pl.BlockSpec((CO, BN), lambda i: (0, i)),
            pl.BlockSpec((CO, (K + 1) * CO), lambda i: (0, 0)),
            pl.BlockSpec((CO, (K + 1) * CO), lambda i: (0, 0)),
            pl.BlockSpec((1, CO), lambda i: (0, 0)),
        ],
        out_specs=pl.BlockSpec((K + 1, BN, CO), lambda i: (0, i, 0)),
        out_shape=jax.ShapeDtypeStruct((K + 1, NPAD, CO), jnp.float32),
    )(z1, xd_pad, w2a, w2b, b2r)


NK16 = CO // 16


def _make_gather(with_stats):
    def body(table_hbm, idx_hbm, *rest):
        if with_stats:
            (out_hbm, parts_hbm, idx_v, buf, sbuf, obuf, stats_v,
             semg0, semg1, sems0, sems1) = rest
        else:
            (out_hbm, idx_v, buf, sbuf, obuf,
             semg0, semg1, sems0, sems1) = rest
        wid = lax.axis_index("s") * NC + lax.axis_index("c")
        base_chunk = wid * NCHUNK
        semg = (semg0, semg1)
        sems = (sems0, sems1)

        def fire(b, g):
            pltpu.sync_copy(idx_hbm.at[base_chunk + g], idx_v.at[b])
            for t in range(K):
                pltpu.async_copy(table_hbm.at[idx_v.at[b, t]], buf.at[b, t],
                                 semg[b])
            pltpu.async_copy(
                table_hbm.at[pl.ds((base_chunk + g) * CHUNK, CHUNK)],
                sbuf.at[b], semg[b])

        def wait_gathers(b):
            for t in range(K):
                pltpu.make_async_copy(table_hbm.at[pl.ds(0, CHUNK)],
                                      buf.at[b, t], semg[b]).wait()
            pltpu.make_async_copy(table_hbm.at[pl.ds(0, CHUNK)],
                                  sbuf.at[b], semg[b]).wait()

        def accumulate(b, g, st):
            def row_body(r, stc):
                out = list(stc) if with_stats else stc
                for k in range(NK16):
                    sl = pl.ds(k * 16, 16)
                    acc = sbuf[b, r, sl] + buf[b, 0, r, sl]
                    for t in range(1, K):
                        acc = acc + buf[b, t, r, sl]
                    obuf[b, r, sl] = acc
                    if with_stats:
                        valid = ((base_chunk + g) * CHUNK + r) < N
                        av = jnp.where(valid, acc, 0.0)
                        out[k] = out[k] + av
                        out[NK16 + k] = out[NK16 + k] + av * av
                return tuple(out) if with_stats else stc

            return lax.fori_loop(0, CHUNK, row_body, st)

        def consume(b, i, st):
            g = 2 * i + b
            wait_gathers(b)

            @pl.when(i > 0)
            def _():
                pltpu.make_async_copy(obuf.at[b],
                                      out_hbm.at[pl.ds(0, CHUNK)],
                                      sems[b]).wait()

            st = accumulate(b, g, st)
            pltpu.async_copy(
                obuf.at[b],
                out_hbm.at[pl.ds((base_chunk + g) * CHUNK, CHUNK)],
                sems[b])

            @pl.when(i < NCHUNK // 2 - 1)
            def _():
                fire(b, g + 2)

            return st

        fire(0, 0)
        fire(1, 1)
        init = (tuple(jnp.zeros((16,), jnp.float32) for _ in range(2 * NK16))
                if with_stats else 0)

        def pair_body(i, st):
            st = consume(0, i, st)
            st = consume(1, i, st)
            return st

        st = lax.fori_loop(0, NCHUNK // 2, pair_body, init)
        for b in range(2):
            pltpu.make_async_copy(obuf.at[b], out_hbm.at[pl.ds(0, CHUNK)],
                                  sems[b]).wait()
        if with_stats:
            for k in range(NK16):
                stats_v[0, pl.ds(k * 16, 16)] = st[k]
                stats_v[1, pl.ds(k * 16, 16)] = st[NK16 + k]
            pltpu.sync_copy(stats_v, parts_hbm.at[wid])

    out_type = jax.ShapeDtypeStruct((NPAD, CO), jnp.float32)
    scratch = [
        pltpu.VMEM((2, K, CHUNK), jnp.int32),
        pltpu.VMEM((2, K, CHUNK, CO), jnp.float32),
        pltpu.VMEM((2, CHUNK, CO), jnp.float32),
        pltpu.VMEM((2, CHUNK, CO), jnp.float32),
    ]
    if with_stats:
        out_type = [out_type,
                    jax.ShapeDtypeStruct((NWORK, 2, CO), jnp.float32)]
        scratch = scratch + [pltpu.VMEM((2, CO), jnp.float32)]
    scratch = scratch + [pltpu.SemaphoreType.DMA] * 4
    return functools.partial(
        pl.kernel,
        mesh=plsc.VectorSubcoreMesh(core_axis_name="c", subcore_axis_name="s"),
        out_type=out_type,
        scratch_types=scratch,
    )(body)


_gather_plain = _make_gather(False)
_gather_stats = _make_gather(True)


def _norm_body(z_ref, p_ref, o_ref):
    z = z_ref[...]
    ssum = jnp.sum(p_ref[:, 0, :], axis=0)
    sqsum = jnp.sum(p_ref[:, 1, :], axis=0)
    mean = ssum * (1.0 / N)
    var = sqsum * (1.0 / N) - mean * mean
    inv = lax.rsqrt(var + EPS)
    zn = (z - mean[None, :]) * inv[None, :]
    o_ref[...] = zn.T[None]


def _norm(z2, parts):
    return pl.pallas_call(
        _norm_body,
        grid=(GRID,),
        in_specs=[
            pl.BlockSpec((BN, CO), lambda i: (i, 0)),
            pl.BlockSpec((NWORK, 2, CO), lambda i: (0, 0, 0)),
        ],
        out_specs=pl.BlockSpec((1, CO, BN), lambda i: (0, 0, i)),
        out_shape=jax.ShapeDtypeStruct((1, CO, N), jnp.float32),
    )(z2, parts)


def kernel(from_up, from_down, neighbors, W1, b1, W2, b2):
    f32 = jnp.float32
    xu = from_up[0]
    xd = from_down[0]

    bf16 = jnp.bfloat16
    w1cat = W1.transpose(1, 2, 0).reshape(CI, (K + 1) * CO).astype(bf16)
    w2a = W2[:, :CO, :].transpose(1, 2, 0).reshape(CO, (K + 1) * CO).astype(bf16)
    w2b = W2[:, CO:, :].transpose(1, 2, 0).reshape(CO, (K + 1) * CO).astype(bf16)
    b1r = b1[None, :].astype(f32)
    b2r = b2[None, :].astype(f32)

    nbr_pad = jnp.concatenate(
        [neighbors.astype(jnp.int32),
         jnp.zeros((NPAD - N, K), jnp.int32)], axis=0)
    adj = nbr_pad + ((jnp.arange(K, dtype=jnp.int32) + 1) * NPAD)[None, :]
    idxarr = adj.reshape(NWORK * NCHUNK, CHUNK, K).transpose(0, 2, 1)

    y1 = _proj1(xu, w1cat, b1r)
    z1 = _gather_plain(y1.reshape((K + 1) * NPAD, CO), idxarr)
    y2 = _proj2(z1, xd, w2a, w2b, b2r)
    z2, parts = _gather_stats(y2.reshape((K + 1) * NPAD, CO), idxarr)
    return _norm(z2, parts)


# speedup vs baseline: 1.8265x; 1.0708x over previous
import functools

import jax
import jax.numpy as jnp
from jax import lax
from jax.experimental import pallas as pl
from jax.experimental.pallas import tpu as pltpu
from jax.experimental.pallas import tpu_sc as plsc

N = 50000
CI = 128
CO = 128
K = 6

NC = 2
NS = 16
NWORK = NC * NS
NPW = 1568
NPAD = NWORK * NPW
CHUNK = 56
NCHUNK = NPW // CHUNK
BN = 1792
GRID = NPAD // BN
EPS = 1e-5


def _proj1_body(x_ref, w_ref, b_ref, y_ref):
    xb = x_ref[...].astype(jnp.bfloat16)
    for j in range(K + 1):
        w = w_ref[:, j * CO:(j + 1) * CO]
        r = lax.dot_general(xb, w, (((0,), (0,)), ((), ())),
                            preferred_element_type=jnp.float32)
        if j == 0:
            y_ref[0, :, :] = r + b_ref[...]
        else:
            y_ref[j, :, :] = r


def _proj1(x_pad, w1cat, b1r):
    return pl.pallas_call(
        _proj1_body,
        grid=(GRID,),
        in_specs=[
            pl.BlockSpec((CI, BN), lambda i: (0, i)),
            pl.BlockSpec((CI, (K + 1) * CO), lambda i: (0, 0)),
            pl.BlockSpec((1, CO), lambda i: (0, 0)),
        ],
        out_specs=pl.BlockSpec((K + 1, BN, CO), lambda i: (0, i, 0)),
        out_shape=jax.ShapeDtypeStruct((K + 1, NPAD, CO), jnp.float32),
    )(x_pad, w1cat, b1r)


def _proj2_body(z1_ref, xd_ref, wa_ref, wb_ref, b_ref, y_ref):
    x1 = z1_ref[...].astype(jnp.bfloat16)
    xd = xd_ref[...].astype(jnp.bfloat16)
    for j in range(K + 1):
        wa = wa_ref[:, j * CO:(j + 1) * CO]
        wb = wb_ref[:, j * CO:(j + 1) * CO]
        r = lax.dot_general(x1, wa, (((1,), (0,)), ((), ())),
                            preferred_element_type=jnp.float32)
        r = r + lax.dot_general(xd, wb, (((0,), (0,)), ((), ())),
                                preferred_element_type=jnp.float32)
        if j == 0:
            y_ref[0, :, :] = r + b_ref[...]
        else:
            y_ref[j, :, :] = r


def _proj2(z1, xd_pad, w2a, w2b, b2r):
    return pl.pallas_call(
        _proj2_body,
        grid=(GRID,),
        in_specs=[
            pl.BlockSpec((BN, CO), lambda i: (i, 0)),
            pl.BlockSpec((CO, BN), lambda i: (0, i)),
            pl.BlockSpec((CO, (K + 1) * CO), lambda i: (0, 0)),
            pl.BlockSpec((CO, (K + 1) * CO), lambda i: (0, 0)),
            pl.BlockSpec((1, CO), lambda i: (0, 0)),
        ],
        out_specs=pl.BlockSpec((K + 1, BN, CO), lambda i: (0, i, 0)),
        out_shape=jax.ShapeDtypeStruct((K + 1, NPAD, CO), jnp.float32),
    )(z1, xd_pad, w2a, w2b, b2r)


NK16 = CO // 16


def _make_gather(with_stats):
    def body(table_hbm, idx_hbm, *rest):
        if with_stats:
            (out_hbm, parts_hbm, idx_v, buf, sbuf, obuf, stats_v,
             semg0, semg1, sems0, sems1) = rest
        else:
            (out_hbm, idx_v, buf, sbuf, obuf,
             semg0, semg1, sems0, sems1) = rest
        wid = lax.axis_index("s") * NC + lax.axis_index("c")
        base_chunk = wid * NCHUNK
        semg = (semg0, semg1)
        sems = (sems0, sems1)

        def fire(b, g):
            pltpu.sync_copy(idx_hbm.at[base_chunk + g], idx_v.at[b])
            for t in range(K):
                pltpu.async_copy(table_hbm.at[idx_v.at[b, t]], buf.at[b, t],
                                 semg[b])
            pltpu.async_copy(
                table_hbm.at[pl.ds((base_chunk + g) * CHUNK, CHUNK)],
                sbuf.at[b], semg[b])

        def wait_gathers(b):
            for t in range(K):
                pltpu.make_async_copy(table_hbm.at[pl.ds(0, CHUNK)],
                                      buf.at[b, t], semg[b]).wait()
            pltpu.make_async_copy(table_hbm.at[pl.ds(0, CHUNK)],
                                  sbuf.at[b], semg[b]).wait()

        def accumulate(b, g, st):
            def row_body(r, stc):
                out = list(stc) if with_stats else stc
                for k in range(NK16):
                    sl = pl.ds(k * 16, 16)
                    acc = sbuf[b, r, sl] + buf[b, 0, r, sl]
                    for t in range(1, K):
                        acc = acc + buf[b, t, r, sl]
                    obuf[b, r, sl] = acc
                    if with_stats:
                        valid = ((base_chunk + g) * CHUNK + r) < N
                        av = jnp.where(valid, acc, 0.0)
                        out[k] = out[k] + av
                        out[NK16 + k] = out[NK16 + k] + av * av
                return tuple(out) if with_stats else stc

            return lax.fori_loop(0, CHUNK, row_body, st)

        def consume(b, i, st):
            g = 2 * i + b
            wait_gathers(b)

            @pl.when(i > 0)
            def _():
                pltpu.make_async_copy(obuf.at[b],
                                      out_hbm.at[pl.ds(0, CHUNK)],
                                      sems[b]).wait()

            st = accumulate(b, g, st)
            pltpu.async_copy(
                obuf.at[b],
                out_hbm.at[pl.ds((base_chunk + g) * CHUNK, CHUNK)],
                sems[b])

            @pl.when(i < NCHUNK // 2 - 1)
            def _():
                fire(b, g + 2)

            return st

        fire(0, 0)
        fire(1, 1)
        init = (tuple(jnp.zeros((16,), jnp.float32) for _ in range(2 * NK16))
                if with_stats else 0)

        def pair_body(i, st):
            st = consume(0, i, st)
            st = consume(1, i, st)
            return st

        st = lax.fori_loop(0, NCHUNK // 2, pair_body, init)
        for b in range(2):
            pltpu.make_async_copy(obuf.at[b], out_hbm.at[pl.ds(0, CHUNK)],
                                  sems[b]).wait()
        if with_stats:
            for k in range(NK16):
                stats_v[0, pl.ds(k * 16, 16)] = st[k]
                stats_v[1, pl.ds(k * 16, 16)] = st[NK16 + k]
            pltpu.sync_copy(stats_v, parts_hbm.at[wid])

    out_type = jax.ShapeDtypeStruct((NPAD, CO), jnp.float32)
    scratch = [
        pltpu.VMEM((2, K, CHUNK), jnp.int32),
        pltpu.VMEM((2, K, CHUNK, CO), jnp.float32),
        pltpu.VMEM((2, CHUNK, CO), jnp.float32),
        pltpu.VMEM((2, CHUNK, CO), jnp.float32),
    ]
    if with_stats:
        out_type = [out_type,
                    jax.ShapeDtypeStruct((NWORK, 2, CO), jnp.float32)]
        scratch = scratch + [pltpu.VMEM((2, CO), jnp.float32)]
    scratch = scratch + [pltpu.SemaphoreType.DMA] * 4
    return functools.partial(
        pl.kernel,
        mesh=plsc.VectorSubcoreMesh(core_axis_name="c", subcore_axis_name="s"),
        out_type=out_type,
        scratch_types=scratch,
    )(body)


_gather_plain = _make_gather(False)
_gather_stats = _make_gather(True)


def _norm_body(z_ref, p_ref, o_ref):
    z = z_ref[...]
    ssum = jnp.sum(p_ref[:, 0, :], axis=0)
    sqsum = jnp.sum(p_ref[:, 1, :], axis=0)
    mean = ssum * (1.0 / N)
    var = sqsum * (1.0 / N) - mean * mean
    inv = lax.rsqrt(var + EPS)
    zn = (z - mean[None, :]) * inv[None, :]
    o_ref[...] = zn.T[None]


def _norm(z2, parts):
    return pl.pallas_call(
        _norm_body,
        grid=(GRID,),
        in_specs=[
            pl.BlockSpec((BN, CO), lambda i: (i, 0)),
            pl.BlockSpec((NWORK, 2, CO), lambda i: (0, 0, 0)),
        ],
        out_specs=pl.BlockSpec((1, CO, BN), lambda i: (0, 0, i)),
        out_shape=jax.ShapeDtypeStruct((1, CO, N), jnp.float32),
    )(z2, parts)


def kernel(from_up, from_down, neighbors, W1, b1, W2, b2):
    f32 = jnp.float32
    xu = from_up[0]
    xd = from_down[0]

    bf16 = jnp.bfloat16
    w1cat = W1.transpose(1, 2, 0).reshape(CI, (K + 1) * CO).astype(bf16)
    w2a = W2[:, :CO, :].transpose(1, 2, 0).reshape(CO, (K + 1) * CO).astype(bf16)
    w2b = W2[:, CO:, :].transpose(1, 2, 0).reshape(CO, (K + 1) * CO).astype(bf16)
    b1r = b1[None, :].astype(f32)
    b2r = b2[None, :].astype(f32)

    nbr_pad = jnp.concatenate(
        [neighbors.astype(jnp.int32),
         jnp.zeros((NPAD - N, K), jnp.int32)], axis=0)
    adj = nbr_pad + ((jnp.arange(K, dtype=jnp.int32) + 1) * NPAD)[None, :]
    idxarr = adj.reshape(NWORK * NCHUNK, CHUNK, K).transpose(0, 2, 1)

    y1 = _proj1(xu, w1cat, b1r)
    z1 = _gather_plain(y1.reshape((K + 1) * NPAD, CO), idxarr)
    y2 = _proj2(z1, xd, w2a, w2b, b2r)
    z2, parts = _gather_stats(y2.reshape((K + 1) * NPAD, CO), idxarr)
    return _norm(z2, parts)


# speedup vs baseline: 1.8881x; 1.0337x over previous
import functools

import jax
import jax.numpy as jnp
from jax import lax
from jax.experimental import pallas as pl
from jax.experimental.pallas import tpu as pltpu
from jax.experimental.pallas import tpu_sc as plsc

N = 50000
CI = 128
CO = 128
K = 6

NC = 2
NS = 16
NWORK = NC * NS
NPW = 1568
NPAD = NWORK * NPW
CHUNK = 56
NCHUNK = NPW // CHUNK
BN = 3584
GRID = NPAD // BN
EPS = 1e-5


def _proj1_body(x_ref, w_ref, b_ref, y_ref):
    xb = x_ref[...].astype(jnp.bfloat16)
    for j in range(K + 1):
        w = w_ref[:, j * CO:(j + 1) * CO]
        r = lax.dot_general(xb, w, (((0,), (0,)), ((), ())),
                            preferred_element_type=jnp.float32)
        if j == 0:
            y_ref[0, :, :] = r + b_ref[...]
        else:
            y_ref[j, :, :] = r


def _proj1(x_pad, w1cat, b1r):
    return pl.pallas_call(
        _proj1_body,
        grid=(GRID,),
        in_specs=[
            pl.BlockSpec((CI, BN), lambda i: (0, i)),
            pl.BlockSpec((CI, (K + 1) * CO), lambda i: (0, 0)),
            pl.BlockSpec((1, CO), lambda i: (0, 0)),
        ],
        out_specs=pl.BlockSpec((K + 1, BN, CO), lambda i: (0, i, 0)),
        out_shape=jax.ShapeDtypeStruct((K + 1, NPAD, CO), jnp.float32),
    )(x_pad, w1cat, b1r)


def _proj2_body(z1_ref, xd_ref, wa_ref, wb_ref, b_ref, y_ref):
    x1 = z1_ref[...].astype(jnp.bfloat16)
    xd = xd_ref[...].astype(jnp.bfloat16)
    for j in range(K + 1):
        wa = wa_ref[:, j * CO:(j + 1) * CO]
        wb = wb_ref[:, j * CO:(j + 1) * CO]
        r = lax.dot_general(x1, wa, (((1,), (0,)), ((), ())),
                            preferred_element_type=jnp.float32)
        r = r + lax.dot_general(xd, wb, (((0,), (0,)), ((), ())),
                                preferred_element_type=jnp.float32)
        if j == 0:
            y_ref[0, :, :] = r + b_ref[...]
        else:
            y_ref[j, :, :] = r


def _proj2(z1, xd_pad, w2a, w2b, b2r):
    return pl.pallas_call(
        _proj2_body,
        grid=(GRID,),
        in_specs=[
            pl.BlockSpec((BN, CO), lambda i: (i, 0)),
            pl.BlockSpec((CO, BN), lambda i: (0, i)),
            pl.BlockSpec((CO, (K + 1) * CO), lambda i: (0, 0)),
            pl.BlockSpec((CO, (K + 1) * CO), lambda i: (0, 0)),
            pl.BlockSpec((1, CO), lambda i: (0, 0)),
        ],
        out_specs=pl.BlockSpec((K + 1, BN, CO), lambda i: (0, i, 0)),
        out_shape=jax.ShapeDtypeStruct((K + 1, NPAD, CO), jnp.float32),
    )(z1, xd_pad, w2a, w2b, b2r)


NK16 = CO // 16


def _make_gather(with_stats):
    def body(table_hbm, idx_hbm, *rest):
        if with_stats:
            (out_hbm, parts_hbm, idx_v, buf, sbuf, obuf, stats_v,
             semg0, semg1, sems0, sems1) = rest
        else:
            (out_hbm, idx_v, buf, sbuf, obuf,
             semg0, semg1, sems0, sems1) = rest
        wid = lax.axis_index("s") * NC + lax.axis_index("c")
        base_chunk = wid * NCHUNK
        semg = (semg0, semg1)
        sems = (sems0, sems1)

        def fire(b, g):
            pltpu.sync_copy(idx_hbm.at[base_chunk + g], idx_v.at[b])
            for t in range(K):
                pltpu.async_copy(table_hbm.at[idx_v.at[b, t]], buf.at[b, t],
                                 semg[b])
            pltpu.async_copy(
                table_hbm.at[pl.ds((base_chunk + g) * CHUNK, CHUNK)],
                sbuf.at[b], semg[b])

        def wait_gathers(b):
            for t in range(K):
                pltpu.make_async_copy(table_hbm.at[pl.ds(0, CHUNK)],
                                      buf.at[b, t], semg[b]).wait()
            pltpu.make_async_copy(table_hbm.at[pl.ds(0, CHUNK)],
                                  sbuf.at[b], semg[b]).wait()

        def accumulate(b, g, st):
            def row_body(r, stc):
                out = list(stc) if with_stats else stc
                for k in range(NK16):
                    sl = pl.ds(k * 16, 16)
                    acc = sbuf[b, r, sl] + buf[b, 0, r, sl]
                    for t in range(1, K):
                        acc = acc + buf[b, t, r, sl]
                    obuf[b, r, sl] = acc
                    if with_stats:
                        valid = ((base_chunk + g) * CHUNK + r) < N
                        av = jnp.where(valid, acc, 0.0)
                        out[k] = out[k] + av
                        out[NK16 + k] = out[NK16 + k] + av * av
                return tuple(out) if with_stats else stc

            return lax.fori_loop(0, CHUNK, row_body, st)

        def consume(b, i, st):
            g = 2 * i + b
            wait_gathers(b)

            @pl.when(i > 0)
            def _():
                pltpu.make_async_copy(obuf.at[b],
                                      out_hbm.at[pl.ds(0, CHUNK)],
                                      sems[b]).wait()

            st = accumulate(b, g, st)
            pltpu.async_copy(
                obuf.at[b],
                out_hbm.at[pl.ds((base_chunk + g) * CHUNK, CHUNK)],
                sems[b])

            @pl.when(i < NCHUNK // 2 - 1)
            def _():
                fire(b, g + 2)

            return st

        fire(0, 0)
        fire(1, 1)
        init = (tuple(jnp.zeros((16,), jnp.float32) for _ in range(2 * NK16))
                if with_stats else 0)

        def pair_body(i, st):
            st = consume(0, i, st)
            st = consume(1, i, st)
            return st

        st = lax.fori_loop(0, NCHUNK // 2, pair_body, init)
        for b in range(2):
            pltpu.make_async_copy(obuf.at[b], out_hbm.at[pl.ds(0, CHUNK)],
                                  sems[b]).wait()
        if with_stats:
            for k in range(NK16):
                stats_v[0, pl.ds(k * 16, 16)] = st[k]
                stats_v[1, pl.ds(k * 16, 16)] = st[NK16 + k]
            pltpu.sync_copy(stats_v, parts_hbm.at[wid])

    out_type = jax.ShapeDtypeStruct((NPAD, CO), jnp.float32)
    scratch = [
        pltpu.VMEM((2, K, CHUNK), jnp.int32),
        pltpu.VMEM((2, K, CHUNK, CO), jnp.float32),
        pltpu.VMEM((2, CHUNK, CO), jnp.float32),
        pltpu.VMEM((2, CHUNK, CO), jnp.float32),
    ]
    if with_stats:
        out_type = [out_type,
                    jax.ShapeDtypeStruct((NWORK, 2, CO), jnp.float32)]
        scratch = scratch + [pltpu.VMEM((2, CO), jnp.float32)]
    scratch = scratch + [pltpu.SemaphoreType.DMA] * 4
    return functools.partial(
        pl.kernel,
        mesh=plsc.VectorSubcoreMesh(core_axis_name="c", subcore_axis_name="s"),
        out_type=out_type,
        scratch_types=scratch,
    )(body)


_gather_plain = _make_gather(False)
_gather_stats = _make_gather(True)


def _norm_body(z_ref, p_ref, o_ref):
    z = z_ref[...]
    ssum = jnp.sum(p_ref[:, 0, :], axis=0)
    sqsum = jnp.sum(p_ref[:, 1, :], axis=0)
    mean = ssum * (1.0 / N)
    var = sqsum * (1.0 / N) - mean * mean
    inv = lax.rsqrt(var + EPS)
    zn = (z - mean[None, :]) * inv[None, :]
    o_ref[...] = zn.T[None]


def _norm(z2, parts):
    return pl.pallas_call(
        _norm_body,
        grid=(GRID,),
        in_specs=[
            pl.BlockSpec((BN, CO), lambda i: (i, 0)),
            pl.BlockSpec((NWORK, 2, CO), lambda i: (0, 0, 0)),
        ],
        out_specs=pl.BlockSpec((1, CO, BN), lambda i: (0, 0, i)),
        out_shape=jax.ShapeDtypeStruct((1, CO, N), jnp.float32),
    )(z2, parts)


def kernel(from_up, from_down, neighbors, W1, b1, W2, b2):
    f32 = jnp.float32
    xu = from_up[0]
    xd = from_down[0]

    bf16 = jnp.bfloat16
    w1cat = W1.transpose(1, 2, 0).reshape(CI, (K + 1) * CO).astype(bf16)
    w2a = W2[:, :CO, :].transpose(1, 2, 0).reshape(CO, (K + 1) * CO).astype(bf16)
    w2b = W2[:, CO:, :].transpose(1, 2, 0).reshape(CO, (K + 1) * CO).astype(bf16)
    b1r = b1[None, :].astype(f32)
    b2r = b2[None, :].astype(f32)

    nbr_pad = jnp.concatenate(
        [neighbors.astype(jnp.int32),
         jnp.zeros((NPAD - N, K), jnp.int32)], axis=0)
    adj = nbr_pad + ((jnp.arange(K, dtype=jnp.int32) + 1) * NPAD)[None, :]
    idxarr = adj.reshape(NWORK * NCHUNK, CHUNK, K).transpose(0, 2, 1)

    y1 = _proj1(xu, w1cat, b1r)
    z1 = _gather_plain(y1.reshape((K + 1) * NPAD, CO), idxarr)
    y2 = _proj2(z1, xd, w2a, w2b, b2r)
    z2, parts = _gather_stats(y2.reshape((K + 1) * NPAD, CO), idxarr)
    return _norm(z2, parts)
